# baseline (speedup 1.0000x reference)
import functools

import jax
import jax.numpy as jnp
import numpy as np
from jax.experimental import pallas as pl
from jax.experimental.pallas import tpu as pltpu

_IGNORE = 255
_THRESH = float(-np.log(0.7))


def _ohem_kernel(*refs, cblk, nbatch, nchunk):
    preds_refs = refs[:cblk]
    labels_ref = refs[cblk]
    out_ref = refs[cblk + 1]
    loss_buf, m_s, s_s, xl_s, acc = refs[cblk + 2:]

    b = pl.program_id(0)
    c = pl.program_id(1)

    @pl.when(jnp.logical_and(b == 0, c == 0))
    def _init_acc():
        acc[0] = 0.0
        acc[1] = 0.0
        acc[2] = 0.0

    @pl.when(c == 0)
    def _init_state():
        m_s[...] = jnp.full(m_s.shape, -jnp.inf, jnp.float32)
        s_s[...] = jnp.zeros(s_s.shape, jnp.float32)
        xl_s[...] = jnp.zeros(xl_s.shape, jnp.float32)

    def tree_reduce(fn, vals):
        while len(vals) > 1:
            vals = [fn(vals[i], vals[i + 1]) if i + 1 < len(vals) else vals[i]
                    for i in range(0, len(vals), 2)]
        return vals[0]

    lbl = labels_ref[0]

    @pl.when(c < nchunk)
    def _chunk_update():
        xs = [r[0, 0] for r in preds_refs]
        m_old = m_s[...]
        m_new = jnp.maximum(m_old, tree_reduce(jnp.maximum, xs))
        csum = tree_reduce(jnp.add, [jnp.exp(x - m_new) for x in xs])
        dlbl = lbl - c * cblk
        tree = list(xs)
        bit = 1
        while len(tree) > 1:
            sel = (dlbl & bit) != 0
            nxt = []
            for i in range(0, len(tree), 2):
                hi = tree[i + 1] if i + 1 < len(tree) else tree[i]
                nxt.append(jnp.where(sel, hi, tree[i]))
            tree = nxt
            bit *= 2
        inchunk = dlbl.astype(jnp.uint32) < jnp.uint32(cblk)
        s_s[...] = s_s[...] * jnp.exp(m_old - m_new) + csum
        xl_s[...] = xl_s[...] + jnp.where(inchunk, tree[0], 0.0)
        m_s[...] = m_new

    @pl.when(c == nchunk)
    def _finish_batch():
        lse = m_s[...] + jnp.log(s_s[...])
        valid = lbl != _IGNORE
        loss = jnp.where(valid, lse - xl_s[...], 0.0)
        loss_buf[b] = loss
        hard = loss > _THRESH
        acc[0] = acc[0] + jnp.sum(valid.astype(jnp.float32))
        acc[1] = acc[1] + jnp.sum(hard.astype(jnp.float32))
        acc[2] = acc[2] + jnp.sum(jnp.where(hard, loss, 0.0))

        @pl.when(b == nbatch - 1)
        def _finish():
            cv = acc[0].astype(jnp.int32)
            nh = acc[1].astype(jnp.int32)
            n_min = cv // 16
            mean_hard = acc[2] / acc[1]

            def topk_mean():
                lb = loss_buf[...]
                bits = jax.lax.bitcast_convert_type(lb, jnp.int32)
                k = n_min

                def body(j, prefix):
                    cand = prefix | (jnp.int32(1) << (30 - j))
                    cnt = jnp.sum((bits >= cand).astype(jnp.int32))
                    return jnp.where(cnt >= k, cand, prefix)

                vbits = jax.lax.fori_loop(0, 31, body, jnp.int32(0))
                v = jax.lax.bitcast_convert_type(vbits, jnp.float32)
                gt = bits > vbits
                cnt_gt = jnp.sum(gt.astype(jnp.int32))
                sum_gt = jnp.sum(jnp.where(gt, lb, 0.0))
                kf = k.astype(jnp.float32)
                return (sum_gt + (kf - cnt_gt.astype(jnp.float32)) * v) / kf

            out_ref[0] = jax.lax.cond(nh < n_min, topk_mean,
                                      lambda: mean_hard)


def kernel(preds, labels):
    B, C, H, W = preds.shape
    R, L = H, W
    cblk = 15 if C % 15 == 0 else C
    nchunk = C // cblk

    def mk_spec(j):
        return pl.BlockSpec(
            (1, 1, R, L),
            lambda b, c, j=j: (b, jnp.minimum(c, nchunk - 1) * cblk + j, 0, 0))

    out = pl.pallas_call(
        functools.partial(_ohem_kernel, cblk=cblk, nbatch=B, nchunk=nchunk),
        grid=(B, nchunk + 1),
        in_specs=[mk_spec(j) for j in range(cblk)]
                 + [pl.BlockSpec((1, R, L), lambda b, c: (b, 0, 0))],
        out_specs=pl.BlockSpec(memory_space=pltpu.SMEM),
        out_shape=jax.ShapeDtypeStruct((1,), jnp.float32),
        scratch_shapes=[
            pltpu.VMEM((B, R, L), jnp.float32),
            pltpu.VMEM((R, L), jnp.float32),
            pltpu.VMEM((R, L), jnp.float32),
            pltpu.VMEM((R, L), jnp.float32),
            pltpu.SMEM((3,), jnp.float32),
        ],
    )(*([preds] * cblk + [labels]))
    return out[0]


# speedup vs baseline: 1.0232x; 1.0232x over previous
import functools

import jax
import jax.numpy as jnp
import numpy as np
from jax.experimental import pallas as pl
from jax.experimental.pallas import tpu as pltpu

_IGNORE = 255
_THRESH = float(-np.log(0.7))


def _ohem_kernel(*refs, cblk, nbatch, nchunk):
    preds_refs = refs[:cblk]
    labels_ref = refs[cblk]
    out_ref = refs[cblk + 1]
    loss_buf, m_s, s_s, xl_s, acc = refs[cblk + 2:]

    b = pl.program_id(0)
    c = pl.program_id(1)

    @pl.when(jnp.logical_and(b == 0, c == 0))
    def _init_acc():
        acc[0] = 0.0
        acc[1] = 0.0
        acc[2] = 0.0

    @pl.when(c == 0)
    def _init_state():
        m_s[...] = jnp.full(m_s.shape, -jnp.inf, jnp.float32)
        s_s[...] = jnp.zeros(s_s.shape, jnp.float32)
        xl_s[...] = jnp.zeros(xl_s.shape, jnp.float32)

    def tree_reduce(fn, vals):
        while len(vals) > 1:
            vals = [fn(vals[i], vals[i + 1]) if i + 1 < len(vals) else vals[i]
                    for i in range(0, len(vals), 2)]
        return vals[0]

    xs = [r[0, 0] for r in preds_refs]
    lbl = labels_ref[0]
    m_old = m_s[...]
    m_new = jnp.maximum(m_old, tree_reduce(jnp.maximum, xs))
    csum = tree_reduce(jnp.add, [jnp.exp(x - m_new) for x in xs])
    dlbl = lbl - c * cblk
    tree = list(xs)
    bit = 1
    while len(tree) > 1:
        sel = (dlbl & bit) != 0
        nxt = []
        for i in range(0, len(tree), 2):
            hi = tree[i + 1] if i + 1 < len(tree) else tree[i]
            nxt.append(jnp.where(sel, hi, tree[i]))
        tree = nxt
        bit *= 2
    inchunk = dlbl.astype(jnp.uint32) < jnp.uint32(cblk)
    s_s[...] = s_s[...] * jnp.exp(m_old - m_new) + csum
    xl_s[...] = xl_s[...] + jnp.where(inchunk, tree[0], 0.0)
    m_s[...] = m_new

    @pl.when(c == nchunk - 1)
    def _finish_batch():
        lse = m_s[...] + jnp.log(s_s[...])
        valid = lbl != _IGNORE
        loss = jnp.where(valid, lse - xl_s[...], 0.0)
        loss_buf[b] = loss
        hard = loss > _THRESH
        acc[0] = acc[0] + jnp.sum(valid.astype(jnp.float32))
        acc[1] = acc[1] + jnp.sum(hard.astype(jnp.float32))
        acc[2] = acc[2] + jnp.sum(jnp.where(hard, loss, 0.0))

        @pl.when(b == nbatch - 1)
        def _finish():
            cv = acc[0].astype(jnp.int32)
            nh = acc[1].astype(jnp.int32)
            n_min = cv // 16
            mean_hard = acc[2] / acc[1]

            def topk_mean():
                lb = loss_buf[...]
                bits = jax.lax.bitcast_convert_type(lb, jnp.int32)
                k = n_min

                def body(j, prefix):
                    cand = prefix | (jnp.int32(1) << (30 - j))
                    cnt = jnp.sum((bits >= cand).astype(jnp.int32))
                    return jnp.where(cnt >= k, cand, prefix)

                vbits = jax.lax.fori_loop(0, 31, body, jnp.int32(0))
                v = jax.lax.bitcast_convert_type(vbits, jnp.float32)
                gt = bits > vbits
                cnt_gt = jnp.sum(gt.astype(jnp.int32))
                sum_gt = jnp.sum(jnp.where(gt, lb, 0.0))
                kf = k.astype(jnp.float32)
                return (sum_gt + (kf - cnt_gt.astype(jnp.float32)) * v) / kf

            out_ref[0] = jax.lax.cond(nh < n_min, topk_mean,
                                      lambda: mean_hard)


def kernel(preds, labels):
    B, C, H, W = preds.shape
    R, L = H, W
    cblk = 15 if C % 15 == 0 else C
    nchunk = C // cblk

    def mk_spec(j):
        return pl.BlockSpec((1, 1, R, L),
                            lambda b, c, j=j: (b, c * cblk + j, 0, 0))

    out = pl.pallas_call(
        functools.partial(_ohem_kernel, cblk=cblk, nbatch=B, nchunk=nchunk),
        grid=(B, nchunk),
        in_specs=[mk_spec(j) for j in range(cblk)]
                 + [pl.BlockSpec((1, R, L), lambda b, c: (b, 0, 0))],
        out_specs=pl.BlockSpec(memory_space=pltpu.SMEM),
        out_shape=jax.ShapeDtypeStruct((1,), jnp.float32),
        scratch_shapes=[
            pltpu.VMEM((B, R, L), jnp.float32),
            pltpu.VMEM((R, L), jnp.float32),
            pltpu.VMEM((R, L), jnp.float32),
            pltpu.VMEM((R, L), jnp.float32),
            pltpu.SMEM((3,), jnp.float32),
        ],
    )(*([preds] * cblk + [labels]))
    return out[0]
